# Initial kernel scaffold; baseline (speedup 1.0000x reference)
import jax
import jax.numpy as jnp
from jax.experimental import pallas as pl


def kernel(rpn_cls_score, rpn_bbox_pred):
    raise NotImplementedError("write your pallas kernel here")


# speedup vs baseline: 10.4541x; 10.4541x over previous
import numpy as np
import jax
import jax.numpy as jnp
from jax.experimental import pallas as pl

_IM_H, _IM_W = 800, 800
_FEAT_STRIDE = 16
_H, _W = _IM_H // _FEAT_STRIDE, _IM_W // _FEAT_STRIDE
_A = 9
_N = _H * _W * _A
_LANES = 128
_ROWS = 176
_NPAD = _ROWS * _LANES
_PRE_NMS = 6000
_POST_NMS = 300
_NMS_THRESH = 0.7
_MIN_SIZE = 16.0
_OUT_ROWS = 304


def _np_base_anchors():
    scales = np.array([8.0, 16.0, 32.0])
    ratios = np.array([0.5, 1.0, 2.0])
    w, h, xc, yc = 16.0, 16.0, 7.5, 7.5
    size = w * h
    size_ratios = size / ratios
    ws = np.round(np.sqrt(size_ratios))
    hs = np.round(ws * ratios)
    ratio_anchors = np.stack([xc - 0.5 * (ws - 1), yc - 0.5 * (hs - 1),
                              xc + 0.5 * (ws - 1), yc + 0.5 * (hs - 1)], axis=1)
    out = []
    for a in ratio_anchors:
        aw = a[2] - a[0] + 1.0
        ah = a[3] - a[1] + 1.0
        axc = a[0] + 0.5 * (aw - 1)
        ayc = a[1] + 0.5 * (ah - 1)
        ws2 = aw * scales
        hs2 = ah * scales
        out.append(np.stack([axc - 0.5 * (ws2 - 1), ayc - 0.5 * (hs2 - 1),
                             axc + 0.5 * (ws2 - 1), ayc + 0.5 * (hs2 - 1)], axis=1))
    return np.concatenate(out, axis=0).astype(np.float32)


def _np_anchor_planes():
    base = _np_base_anchors()
    sx = np.arange(_W) * _FEAT_STRIDE
    sy = np.arange(_H) * _FEAT_STRIDE
    SX, SY = np.meshgrid(sx, sy)
    shifts = np.stack([SX.ravel(), SY.ravel(), SX.ravel(), SY.ravel()],
                      axis=1).astype(np.float32)
    anchors = (shifts[:, None, :] + base[None, :, :]).reshape(-1, 4)
    widths = anchors[:, 2] - anchors[:, 0] + np.float32(1.0)
    heights = anchors[:, 3] - anchors[:, 1] + np.float32(1.0)
    ctr_x = anchors[:, 0] + np.float32(0.5) * widths
    ctr_y = anchors[:, 1] + np.float32(0.5) * heights

    def plane(v, fill):
        out = np.full((_NPAD,), fill, np.float32)
        out[:_N] = v
        return out.reshape(_ROWS, _LANES)

    return (plane(widths, 1.0), plane(heights, 1.0),
            plane(ctr_x, 0.0), plane(ctr_y, 0.0))


_AW, _AH, _ACX, _ACY = _np_anchor_planes()


def _body(a_ref, b_ref, dx_ref, dy_ref, dw_ref, dh_ref,
          aw_ref, ah_ref, acx_ref, acy_ref, out_ref):
    a = a_ref[:]
    b = b_ref[:]
    m = jnp.maximum(a, b)
    ea = jnp.exp(a - m)
    eb = jnp.exp(b - m)
    sc = eb / (ea + eb)

    wdt = aw_ref[:]
    hgt = ah_ref[:]
    cx = acx_ref[:]
    cy = acy_ref[:]
    dx = dx_ref[:]
    dy = dy_ref[:]
    dw = dw_ref[:]
    dh = dh_ref[:]
    pcx = dx * wdt + cx
    pcy = dy * hgt + cy
    pw = jnp.exp(dw) * wdt
    ph = jnp.exp(dh) * hgt
    x1 = jnp.clip(pcx - 0.5 * pw, 0.0, _IM_W - 1.0)
    y1 = jnp.clip(pcy - 0.5 * ph, 0.0, _IM_H - 1.0)
    x2 = jnp.clip(pcx + 0.5 * pw, 0.0, _IM_W - 1.0)
    y2 = jnp.clip(pcy + 0.5 * ph, 0.0, _IM_H - 1.0)
    ws = x2 - x1 + 1.0
    hs = y2 - y1 + 1.0
    valid = (ws >= _MIN_SIZE) & (hs >= _MIN_SIZE)
    sc = jnp.where(valid, sc, -1e9)

    rows = jax.lax.broadcasted_iota(jnp.int32, (_ROWS, _LANES), 0)
    cols = jax.lax.broadcasted_iota(jnp.int32, (_ROWS, _LANES), 1)
    flat = rows * _LANES + cols
    sc = jnp.where(flat < _N, sc, -jnp.inf)

    kraw = jax.lax.bitcast_convert_type(sc, jnp.int32)
    keys = jnp.where(kraw < 0, kraw ^ jnp.int32(0x7FFFFFFF), kraw)

    def bs_body(k, t):
        cand = t + jnp.left_shift(jnp.int32(1), 30 - k)
        cnt = jnp.sum((keys >= cand).astype(jnp.int32))
        return jnp.where(cnt >= _PRE_NMS, cand, t)

    thr = jax.lax.fori_loop(0, 31, bs_body, jnp.int32(-2147483648))

    s0 = jnp.where(keys >= thr, sc, -jnp.inf)
    areas = ws * hs
    neg = jnp.float32(-jnp.inf)
    li = jax.lax.broadcasted_iota(jnp.int32, (1, _LANES), 1)

    def nms_body(i, carry):
        s, f0 = carry
        mx = jnp.max(s)
        eq = s == mx
        mi = jnp.min(jnp.where(eq, flat, jnp.int32(2147483647)))
        oh = flat == mi
        bx1 = jnp.sum(jnp.where(oh, x1, 0.0))
        by1 = jnp.sum(jnp.where(oh, y1, 0.0))
        bx2 = jnp.sum(jnp.where(oh, x2, 0.0))
        by2 = jnp.sum(jnp.where(oh, y2, 0.0))
        bar = jnp.sum(jnp.where(oh, areas, 0.0))
        pick = jnp.stack([bx1, by1, bx2, by2, bar])
        f0 = jnp.where(i == 0, pick, f0)
        pick = jnp.where(mx == neg, f0, pick)
        bx1, by1, bx2, by2, bar = pick[0], pick[1], pick[2], pick[3], pick[4]
        row = jnp.where(li == 0, bx1,
                        jnp.where(li == 1, by1,
                                  jnp.where(li == 2, bx2,
                                            jnp.where(li == 3, by2, 0.0))))
        out_ref[pl.ds(i, 1), :] = row
        xx1 = jnp.maximum(bx1, x1)
        yy1 = jnp.maximum(by1, y1)
        xx2 = jnp.minimum(bx2, x2)
        yy2 = jnp.minimum(by2, y2)
        iw = jnp.maximum(0.0, xx2 - xx1 + 1.0)
        ih = jnp.maximum(0.0, yy2 - yy1 + 1.0)
        inter = iw * ih
        iou = inter / (bar + areas - inter)
        s = jnp.where(iou > _NMS_THRESH, neg, s)
        return s, f0

    jax.lax.fori_loop(0, _POST_NMS, nms_body, (s0, jnp.zeros((5,), jnp.float32)))


def _plane(v, pad):
    return jnp.concatenate([v, jnp.zeros((pad,), v.dtype)]).reshape(_ROWS, _LANES)


def kernel(rpn_cls_score, rpn_bbox_pred):
    cls = rpn_cls_score.reshape(_N, 2)
    deltas = rpn_bbox_pred.reshape(_N, 4)
    pad = _NPAD - _N
    args = (_plane(cls[:, 0], pad), _plane(cls[:, 1], pad),
            _plane(deltas[:, 0], pad), _plane(deltas[:, 1], pad),
            _plane(deltas[:, 2], pad), _plane(deltas[:, 3], pad),
            _AW, _AH, _ACX, _ACY)
    res = pl.pallas_call(
        _body,
        out_shape=jax.ShapeDtypeStruct((_OUT_ROWS, _LANES), jnp.float32),
    )(*args)
    boxes = res[:_POST_NMS, :4]
    return jnp.concatenate(
        [jnp.zeros((_POST_NMS, 1), jnp.float32), boxes], axis=1)


# speedup vs baseline: 10.8965x; 1.0423x over previous
import numpy as np
import jax
import jax.numpy as jnp
from jax.experimental import pallas as pl
from jax.experimental.pallas import tpu as pltpu

_IM_H, _IM_W = 800, 800
_FEAT_STRIDE = 16
_H, _W = _IM_H // _FEAT_STRIDE, _IM_W // _FEAT_STRIDE
_A = 9
_N = _H * _W * _A
_LANES = 128
_ROWS = 176
_NPAD = _ROWS * _LANES
_PRE_NMS = 6000
_POST_NMS = 300
_NMS_THRESH = 0.7
_MIN_SIZE = 16.0
_OUT_ROWS = 304


def _np_base_anchors():
    scales = np.array([8.0, 16.0, 32.0])
    ratios = np.array([0.5, 1.0, 2.0])
    w, h, xc, yc = 16.0, 16.0, 7.5, 7.5
    size = w * h
    size_ratios = size / ratios
    ws = np.round(np.sqrt(size_ratios))
    hs = np.round(ws * ratios)
    ratio_anchors = np.stack([xc - 0.5 * (ws - 1), yc - 0.5 * (hs - 1),
                              xc + 0.5 * (ws - 1), yc + 0.5 * (hs - 1)], axis=1)
    out = []
    for a in ratio_anchors:
        aw = a[2] - a[0] + 1.0
        ah = a[3] - a[1] + 1.0
        axc = a[0] + 0.5 * (aw - 1)
        ayc = a[1] + 0.5 * (ah - 1)
        ws2 = aw * scales
        hs2 = ah * scales
        out.append(np.stack([axc - 0.5 * (ws2 - 1), ayc - 0.5 * (hs2 - 1),
                             axc + 0.5 * (ws2 - 1), ayc + 0.5 * (hs2 - 1)], axis=1))
    return np.concatenate(out, axis=0).astype(np.float32)


def _np_anchor_planes():
    base = _np_base_anchors()
    sx = np.arange(_W) * _FEAT_STRIDE
    sy = np.arange(_H) * _FEAT_STRIDE
    SX, SY = np.meshgrid(sx, sy)
    shifts = np.stack([SX.ravel(), SY.ravel(), SX.ravel(), SY.ravel()],
                      axis=1).astype(np.float32)
    anchors = (shifts[:, None, :] + base[None, :, :]).reshape(-1, 4)
    widths = anchors[:, 2] - anchors[:, 0] + np.float32(1.0)
    heights = anchors[:, 3] - anchors[:, 1] + np.float32(1.0)
    ctr_x = anchors[:, 0] + np.float32(0.5) * widths
    ctr_y = anchors[:, 1] + np.float32(0.5) * heights

    def plane(v, fill):
        out = np.full((_NPAD,), fill, np.float32)
        out[:_N] = v
        return out.reshape(_ROWS, _LANES)

    return (plane(widths, 1.0), plane(heights, 1.0),
            plane(ctr_x, 0.0), plane(ctr_y, 0.0))


_AW, _AH, _ACX, _ACY = _np_anchor_planes()


def _body(a_ref, b_ref, dx_ref, dy_ref, dw_ref, dh_ref,
          aw_ref, ah_ref, acx_ref, acy_ref, out_ref,
          s_ref, x1_s, y1_s, x2_s, y2_s, ar_s):
    a = a_ref[:]
    b = b_ref[:]
    m = jnp.maximum(a, b)
    ea = jnp.exp(a - m)
    eb = jnp.exp(b - m)
    sc = eb / (ea + eb)

    wdt = aw_ref[:]
    hgt = ah_ref[:]
    cx = acx_ref[:]
    cy = acy_ref[:]
    dx = dx_ref[:]
    dy = dy_ref[:]
    dw = dw_ref[:]
    dh = dh_ref[:]
    pcx = dx * wdt + cx
    pcy = dy * hgt + cy
    pw = jnp.exp(dw) * wdt
    ph = jnp.exp(dh) * hgt
    x1 = jnp.clip(pcx - 0.5 * pw, 0.0, _IM_W - 1.0)
    y1 = jnp.clip(pcy - 0.5 * ph, 0.0, _IM_H - 1.0)
    x2 = jnp.clip(pcx + 0.5 * pw, 0.0, _IM_W - 1.0)
    y2 = jnp.clip(pcy + 0.5 * ph, 0.0, _IM_H - 1.0)
    ws = x2 - x1 + 1.0
    hs = y2 - y1 + 1.0
    valid = (ws >= _MIN_SIZE) & (hs >= _MIN_SIZE)
    sc = jnp.where(valid, sc, -1e9)

    rows = jax.lax.broadcasted_iota(jnp.int32, (_ROWS, _LANES), 0)
    cols = jax.lax.broadcasted_iota(jnp.int32, (_ROWS, _LANES), 1)
    flat = rows * _LANES + cols
    sc = jnp.where(flat < _N, sc, -jnp.inf)

    kraw = jax.lax.bitcast_convert_type(sc, jnp.int32)
    keys = jnp.where(kraw < 0, kraw ^ jnp.int32(0x7FFFFFFF), kraw)

    def bs_body(k, t):
        cand = t + jnp.left_shift(jnp.int32(1), 30 - k)
        cnt = jnp.sum((keys >= cand).astype(jnp.int32))
        return jnp.where(cnt >= _PRE_NMS, cand, t)

    thr = jax.lax.fori_loop(0, 31, bs_body, jnp.int32(-2147483648))

    s_ref[:] = jnp.where(keys >= thr, sc, -jnp.inf)
    areas = ws * hs
    x1_s[:] = x1
    y1_s[:] = y1
    x2_s[:] = x2
    y2_s[:] = y2
    ar_s[:] = areas
    neg = jnp.float32(-jnp.inf)
    li = jax.lax.broadcasted_iota(jnp.int32, (1, _LANES), 1)

    def nms_body(i, f0):
        s = s_ref[:]
        mx = jnp.max(s)
        eq = s == mx
        mi = jnp.min(jnp.where(eq, flat, jnp.int32(2147483647)))
        ri = jax.lax.shift_right_logical(mi, 7)
        ci = jax.lax.bitwise_and(mi, 127)
        oh = li == ci
        bx1 = jnp.sum(jnp.where(oh, x1_s[pl.ds(ri, 1), :], 0.0))
        by1 = jnp.sum(jnp.where(oh, y1_s[pl.ds(ri, 1), :], 0.0))
        bx2 = jnp.sum(jnp.where(oh, x2_s[pl.ds(ri, 1), :], 0.0))
        by2 = jnp.sum(jnp.where(oh, y2_s[pl.ds(ri, 1), :], 0.0))
        bar = jnp.sum(jnp.where(oh, ar_s[pl.ds(ri, 1), :], 0.0))
        pick = jnp.stack([bx1, by1, bx2, by2, bar])
        f0 = jnp.where(i == 0, pick, f0)
        pick = jnp.where(mx == neg, f0, pick)
        bx1, by1, bx2, by2, bar = pick[0], pick[1], pick[2], pick[3], pick[4]
        row = jnp.where(li == 0, bx1,
                        jnp.where(li == 1, by1,
                                  jnp.where(li == 2, bx2,
                                            jnp.where(li == 3, by2, 0.0))))
        out_ref[pl.ds(i, 1), :] = row
        xx1 = jnp.maximum(bx1, x1)
        yy1 = jnp.maximum(by1, y1)
        xx2 = jnp.minimum(bx2, x2)
        yy2 = jnp.minimum(by2, y2)
        iw = jnp.maximum(0.0, xx2 - xx1 + 1.0)
        ih = jnp.maximum(0.0, yy2 - yy1 + 1.0)
        inter = iw * ih
        iou = inter / (bar + areas - inter)
        s_ref[:] = jnp.where(iou > _NMS_THRESH, neg, s)
        return f0

    jax.lax.fori_loop(0, _POST_NMS, nms_body, jnp.zeros((5,), jnp.float32))


def _plane(v, pad):
    return jnp.concatenate([v, jnp.zeros((pad,), v.dtype)]).reshape(_ROWS, _LANES)


def kernel(rpn_cls_score, rpn_bbox_pred):
    cls = rpn_cls_score.reshape(_N, 2)
    deltas = rpn_bbox_pred.reshape(_N, 4)
    pad = _NPAD - _N
    args = (_plane(cls[:, 0], pad), _plane(cls[:, 1], pad),
            _plane(deltas[:, 0], pad), _plane(deltas[:, 1], pad),
            _plane(deltas[:, 2], pad), _plane(deltas[:, 3], pad),
            _AW, _AH, _ACX, _ACY)
    res = pl.pallas_call(
        _body,
        out_shape=jax.ShapeDtypeStruct((_OUT_ROWS, _LANES), jnp.float32),
        scratch_shapes=[pltpu.VMEM((_ROWS, _LANES), jnp.float32)
                        for _ in range(6)],
    )(*args)
    boxes = res[:_POST_NMS, :4]
    return jnp.concatenate(
        [jnp.zeros((_POST_NMS, 1), jnp.float32), boxes], axis=1)


# speedup vs baseline: 12.4601x; 1.1435x over previous
import numpy as np
import jax
import jax.numpy as jnp
from jax.experimental import pallas as pl
from jax.experimental.pallas import tpu as pltpu

_IM_H, _IM_W = 800, 800
_FEAT_STRIDE = 16
_H, _W = _IM_H // _FEAT_STRIDE, _IM_W // _FEAT_STRIDE
_A = 9
_N = _H * _W * _A
_LANES = 128
_ROWS = 176
_NPAD = _ROWS * _LANES
_PRE_NMS = 6000
_POST_NMS = 300
_NMS_THRESH = 0.7
_MIN_SIZE = 16.0
_OUT_ROWS = 304


def _np_base_anchors():
    scales = np.array([8.0, 16.0, 32.0])
    ratios = np.array([0.5, 1.0, 2.0])
    w, h, xc, yc = 16.0, 16.0, 7.5, 7.5
    size = w * h
    size_ratios = size / ratios
    ws = np.round(np.sqrt(size_ratios))
    hs = np.round(ws * ratios)
    ratio_anchors = np.stack([xc - 0.5 * (ws - 1), yc - 0.5 * (hs - 1),
                              xc + 0.5 * (ws - 1), yc + 0.5 * (hs - 1)], axis=1)
    out = []
    for a in ratio_anchors:
        aw = a[2] - a[0] + 1.0
        ah = a[3] - a[1] + 1.0
        axc = a[0] + 0.5 * (aw - 1)
        ayc = a[1] + 0.5 * (ah - 1)
        ws2 = aw * scales
        hs2 = ah * scales
        out.append(np.stack([axc - 0.5 * (ws2 - 1), ayc - 0.5 * (hs2 - 1),
                             axc + 0.5 * (ws2 - 1), ayc + 0.5 * (hs2 - 1)], axis=1))
    return np.concatenate(out, axis=0).astype(np.float32)


def _np_anchor_planes():
    base = _np_base_anchors()
    sx = np.arange(_W) * _FEAT_STRIDE
    sy = np.arange(_H) * _FEAT_STRIDE
    SX, SY = np.meshgrid(sx, sy)
    shifts = np.stack([SX.ravel(), SY.ravel(), SX.ravel(), SY.ravel()],
                      axis=1).astype(np.float32)
    anchors = (shifts[:, None, :] + base[None, :, :]).reshape(-1, 4)
    widths = anchors[:, 2] - anchors[:, 0] + np.float32(1.0)
    heights = anchors[:, 3] - anchors[:, 1] + np.float32(1.0)
    ctr_x = anchors[:, 0] + np.float32(0.5) * widths
    ctr_y = anchors[:, 1] + np.float32(0.5) * heights

    def plane(v, fill):
        out = np.full((_NPAD,), fill, np.float32)
        out[:_N] = v
        return out.reshape(_ROWS, _LANES)

    return (plane(widths, 1.0), plane(heights, 1.0),
            plane(ctr_x, 0.0), plane(ctr_y, 0.0))


_AW, _AH, _ACX, _ACY = _np_anchor_planes()


def _body(a_ref, b_ref, dx_ref, dy_ref, dw_ref, dh_ref,
          aw_ref, ah_ref, acx_ref, acy_ref, out_ref,
          s_ref, x1_s, y1_s, x2_s, y2_s, ar_s):
    a = a_ref[:]
    b = b_ref[:]
    m = jnp.maximum(a, b)
    ea = jnp.exp(a - m)
    eb = jnp.exp(b - m)
    sc = eb / (ea + eb)

    wdt = aw_ref[:]
    hgt = ah_ref[:]
    cx = acx_ref[:]
    cy = acy_ref[:]
    dx = dx_ref[:]
    dy = dy_ref[:]
    dw = dw_ref[:]
    dh = dh_ref[:]
    pcx = dx * wdt + cx
    pcy = dy * hgt + cy
    pw = jnp.exp(dw) * wdt
    ph = jnp.exp(dh) * hgt
    x1 = jnp.clip(pcx - 0.5 * pw, 0.0, _IM_W - 1.0)
    y1 = jnp.clip(pcy - 0.5 * ph, 0.0, _IM_H - 1.0)
    x2 = jnp.clip(pcx + 0.5 * pw, 0.0, _IM_W - 1.0)
    y2 = jnp.clip(pcy + 0.5 * ph, 0.0, _IM_H - 1.0)
    ws = x2 - x1 + 1.0
    hs = y2 - y1 + 1.0
    valid = (ws >= _MIN_SIZE) & (hs >= _MIN_SIZE)
    sc = jnp.where(valid, sc, -1e9)

    rows = jax.lax.broadcasted_iota(jnp.int32, (_ROWS, _LANES), 0)
    cols = jax.lax.broadcasted_iota(jnp.int32, (_ROWS, _LANES), 1)
    flat = rows * _LANES + cols
    sc = jnp.where(flat < _N, sc, -jnp.inf)

    kraw = jax.lax.bitcast_convert_type(sc, jnp.int32)
    keys = jnp.where(kraw < 0, kraw ^ jnp.int32(0x7FFFFFFF), kraw)

    cnt_pos = jnp.sum((keys >= 0).astype(jnp.int32))
    t0 = jnp.where(cnt_pos >= _PRE_NMS, jnp.int32(0), jnp.int32(-2147483648))

    def bs_body(k, t):
        cand = t + jnp.left_shift(jnp.int32(1), 30 - k)
        cnt = jnp.sum((keys >= cand).astype(jnp.int32))
        return jnp.where(cnt >= _PRE_NMS, cand, t)

    thr = jax.lax.fori_loop(0, 31, bs_body, t0)

    s_ref[:] = jnp.where(keys >= thr, sc, -jnp.inf)
    areas = ws * hs
    x1_s[:] = x1
    y1_s[:] = y1
    x2_s[:] = x2
    y2_s[:] = y2
    ar_s[:] = areas
    neg = jnp.float32(-jnp.inf)
    li = jax.lax.broadcasted_iota(jnp.int32, (1, _LANES), 1)

    def nms_body(i, f0):
        f0x1, f0y1, f0x2, f0y2, f0ar = f0
        s = s_ref[:]
        mx = jnp.max(s)
        eq = s == mx
        mi = jnp.min(jnp.where(eq, flat, jnp.int32(2147483647)))
        ri = jax.lax.shift_right_logical(mi, 7)
        ci = jax.lax.bitwise_and(mi, 127)
        oh = li == ci
        bx1 = jnp.sum(jnp.where(oh, x1_s[pl.ds(ri, 1), :], 0.0))
        by1 = jnp.sum(jnp.where(oh, y1_s[pl.ds(ri, 1), :], 0.0))
        bx2 = jnp.sum(jnp.where(oh, x2_s[pl.ds(ri, 1), :], 0.0))
        by2 = jnp.sum(jnp.where(oh, y2_s[pl.ds(ri, 1), :], 0.0))
        bar = jnp.sum(jnp.where(oh, ar_s[pl.ds(ri, 1), :], 0.0))
        first = i == 0
        f0x1 = jnp.where(first, bx1, f0x1)
        f0y1 = jnp.where(first, by1, f0y1)
        f0x2 = jnp.where(first, bx2, f0x2)
        f0y2 = jnp.where(first, by2, f0y2)
        f0ar = jnp.where(first, bar, f0ar)
        dead = mx == neg
        bx1 = jnp.where(dead, f0x1, bx1)
        by1 = jnp.where(dead, f0y1, by1)
        bx2 = jnp.where(dead, f0x2, bx2)
        by2 = jnp.where(dead, f0y2, by2)
        bar = jnp.where(dead, f0ar, bar)
        row = jnp.where(li == 0, bx1,
                        jnp.where(li == 1, by1,
                                  jnp.where(li == 2, bx2,
                                            jnp.where(li == 3, by2, 0.0))))
        out_ref[pl.ds(i, 1), :] = row
        xx1 = jnp.maximum(bx1, x1)
        yy1 = jnp.maximum(by1, y1)
        xx2 = jnp.minimum(bx2, x2)
        yy2 = jnp.minimum(by2, y2)
        iw = jnp.maximum(0.0, xx2 - xx1 + 1.0)
        ih = jnp.maximum(0.0, yy2 - yy1 + 1.0)
        inter = iw * ih
        iou = inter / (bar + areas - inter)
        s_ref[:] = jnp.where(iou > _NMS_THRESH, neg, s)
        return (f0x1, f0y1, f0x2, f0y2, f0ar)

    zero = jnp.float32(0.0)
    jax.lax.fori_loop(0, _POST_NMS, nms_body,
                      (zero, zero, zero, zero, zero))


def _plane(v, pad):
    return jnp.concatenate([v, jnp.zeros((pad,), v.dtype)]).reshape(_ROWS, _LANES)


def kernel(rpn_cls_score, rpn_bbox_pred):
    cls = rpn_cls_score.reshape(_N, 2)
    deltas = rpn_bbox_pred.reshape(_N, 4)
    pad = _NPAD - _N
    args = (_plane(cls[:, 0], pad), _plane(cls[:, 1], pad),
            _plane(deltas[:, 0], pad), _plane(deltas[:, 1], pad),
            _plane(deltas[:, 2], pad), _plane(deltas[:, 3], pad),
            _AW, _AH, _ACX, _ACY)
    res = pl.pallas_call(
        _body,
        out_shape=jax.ShapeDtypeStruct((_OUT_ROWS, _LANES), jnp.float32),
        scratch_shapes=[pltpu.VMEM((_ROWS, _LANES), jnp.float32)
                        for _ in range(6)],
    )(*args)
    boxes = res[:_POST_NMS, :4]
    return jnp.concatenate(
        [jnp.zeros((_POST_NMS, 1), jnp.float32), boxes], axis=1)


# speedup vs baseline: 13.7702x; 1.1051x over previous
import numpy as np
import jax
import jax.numpy as jnp
from jax.experimental import pallas as pl
from jax.experimental.pallas import tpu as pltpu

_IM_H, _IM_W = 800, 800
_FEAT_STRIDE = 16
_H, _W = _IM_H // _FEAT_STRIDE, _IM_W // _FEAT_STRIDE
_A = 9
_N = _H * _W * _A
_LANES = 128
_ROWS = 176
_NPAD = _ROWS * _LANES
_PRE_NMS = 6000
_POST_NMS = 300
_NMS_THRESH = 0.7
_MIN_SIZE = 16.0
_OUT_ROWS = 304


def _np_base_anchors():
    scales = np.array([8.0, 16.0, 32.0])
    ratios = np.array([0.5, 1.0, 2.0])
    w, h, xc, yc = 16.0, 16.0, 7.5, 7.5
    size = w * h
    size_ratios = size / ratios
    ws = np.round(np.sqrt(size_ratios))
    hs = np.round(ws * ratios)
    ratio_anchors = np.stack([xc - 0.5 * (ws - 1), yc - 0.5 * (hs - 1),
                              xc + 0.5 * (ws - 1), yc + 0.5 * (hs - 1)], axis=1)
    out = []
    for a in ratio_anchors:
        aw = a[2] - a[0] + 1.0
        ah = a[3] - a[1] + 1.0
        axc = a[0] + 0.5 * (aw - 1)
        ayc = a[1] + 0.5 * (ah - 1)
        ws2 = aw * scales
        hs2 = ah * scales
        out.append(np.stack([axc - 0.5 * (ws2 - 1), ayc - 0.5 * (hs2 - 1),
                             axc + 0.5 * (ws2 - 1), ayc + 0.5 * (hs2 - 1)], axis=1))
    return np.concatenate(out, axis=0).astype(np.float32)


def _np_anchor_planes():
    base = _np_base_anchors()
    sx = np.arange(_W) * _FEAT_STRIDE
    sy = np.arange(_H) * _FEAT_STRIDE
    SX, SY = np.meshgrid(sx, sy)
    shifts = np.stack([SX.ravel(), SY.ravel(), SX.ravel(), SY.ravel()],
                      axis=1).astype(np.float32)
    anchors = (shifts[:, None, :] + base[None, :, :]).reshape(-1, 4)
    widths = anchors[:, 2] - anchors[:, 0] + np.float32(1.0)
    heights = anchors[:, 3] - anchors[:, 1] + np.float32(1.0)
    ctr_x = anchors[:, 0] + np.float32(0.5) * widths
    ctr_y = anchors[:, 1] + np.float32(0.5) * heights

    def plane(v, fill):
        out = np.full((_NPAD,), fill, np.float32)
        out[:_N] = v
        return out.reshape(_ROWS, _LANES)

    return (plane(widths, 1.0), plane(heights, 1.0),
            plane(ctr_x, 0.0), plane(ctr_y, 0.0))


_AW, _AH, _ACX, _ACY = _np_anchor_planes()


def _body(a_ref, b_ref, dx_ref, dy_ref, dw_ref, dh_ref,
          aw_ref, ah_ref, acx_ref, acy_ref, out_ref,
          s_ref, x1_s, y1_s, x2_s, y2_s, ar_s):
    a = a_ref[:]
    b = b_ref[:]
    m = jnp.maximum(a, b)
    ea = jnp.exp(a - m)
    eb = jnp.exp(b - m)
    sc = eb / (ea + eb)

    wdt = aw_ref[:]
    hgt = ah_ref[:]
    cx = acx_ref[:]
    cy = acy_ref[:]
    dx = dx_ref[:]
    dy = dy_ref[:]
    dw = dw_ref[:]
    dh = dh_ref[:]
    pcx = dx * wdt + cx
    pcy = dy * hgt + cy
    pw = jnp.exp(dw) * wdt
    ph = jnp.exp(dh) * hgt
    x1 = jnp.clip(pcx - 0.5 * pw, 0.0, _IM_W - 1.0)
    y1 = jnp.clip(pcy - 0.5 * ph, 0.0, _IM_H - 1.0)
    x2 = jnp.clip(pcx + 0.5 * pw, 0.0, _IM_W - 1.0)
    y2 = jnp.clip(pcy + 0.5 * ph, 0.0, _IM_H - 1.0)
    ws = x2 - x1 + 1.0
    hs = y2 - y1 + 1.0
    valid = (ws >= _MIN_SIZE) & (hs >= _MIN_SIZE)
    sc = jnp.where(valid, sc, -1e9)

    rows = jax.lax.broadcasted_iota(jnp.int32, (_ROWS, _LANES), 0)
    cols = jax.lax.broadcasted_iota(jnp.int32, (_ROWS, _LANES), 1)
    flat = rows * _LANES + cols
    sc = jnp.where(flat < _N, sc, -jnp.inf)

    kraw = jax.lax.bitcast_convert_type(sc, jnp.int32)
    keys = jnp.where(kraw < 0, kraw ^ jnp.int32(0x7FFFFFFF), kraw)

    cnt_pos = jnp.sum((keys >= 0).astype(jnp.int32))
    t0 = jnp.where(cnt_pos >= _PRE_NMS, jnp.int32(0), jnp.int32(-2147483648))

    def bs_body(k, t):
        cand = t + jnp.left_shift(jnp.int32(1), 30 - k)
        cnt = jnp.sum((keys >= cand).astype(jnp.int32))
        return jnp.where(cnt >= _PRE_NMS, cand, t)

    thr = jax.lax.fori_loop(0, 31, bs_body, t0)

    s_ref[:] = jnp.where(keys >= thr, sc, -jnp.inf)
    areas = ws * hs
    x1_s[:] = x1
    y1_s[:] = y1
    x2_s[:] = x2
    y2_s[:] = y2
    ar_s[:] = areas
    neg = jnp.float32(-jnp.inf)
    li = jax.lax.broadcasted_iota(jnp.int32, (1, _LANES), 1)

    imax = jnp.int32(2147483647)

    def _extract(mi):
        ri = jax.lax.shift_right_logical(mi, 7)
        ci = jax.lax.bitwise_and(mi, 127)
        oh = li == ci
        px1 = jnp.sum(jnp.where(oh, x1_s[pl.ds(ri, 1), :], 0.0))
        py1 = jnp.sum(jnp.where(oh, y1_s[pl.ds(ri, 1), :], 0.0))
        px2 = jnp.sum(jnp.where(oh, x2_s[pl.ds(ri, 1), :], 0.0))
        py2 = jnp.sum(jnp.where(oh, y2_s[pl.ds(ri, 1), :], 0.0))
        par = jnp.sum(jnp.where(oh, ar_s[pl.ds(ri, 1), :], 0.0))
        return px1, py1, px2, py2, par

    def _rowof(px1, py1, px2, py2):
        return jnp.where(li == 0, px1,
                         jnp.where(li == 1, py1,
                                   jnp.where(li == 2, px2,
                                             jnp.where(li == 3, py2, 0.0))))

    def nms_cond(carry):
        return carry[0] < _POST_NMS

    def nms_trip(carry):
        k, f0x1, f0y1, f0x2, f0y2, f0ar = carry
        s = s_ref[:]
        mx = jnp.max(s)
        mi = jnp.min(jnp.where(s == mx, flat, imax))
        ax1, ay1, ax2, ay2, aar = _extract(mi)
        first = k == 0
        f0x1 = jnp.where(first, ax1, f0x1)
        f0y1 = jnp.where(first, ay1, f0y1)
        f0x2 = jnp.where(first, ax2, f0x2)
        f0y2 = jnp.where(first, ay2, f0y2)
        f0ar = jnp.where(first, aar, f0ar)
        deadA = mx == neg
        ax1 = jnp.where(deadA, f0x1, ax1)
        ay1 = jnp.where(deadA, f0y1, ay1)
        ax2 = jnp.where(deadA, f0x2, ax2)
        ay2 = jnp.where(deadA, f0y2, ay2)
        aar = jnp.where(deadA, f0ar, aar)
        sB = jnp.where(flat == mi, neg, s)
        mxB = jnp.max(sB)
        miB = jnp.min(jnp.where(sB == mxB, flat, imax))
        bx1, by1, bx2, by2, bar = _extract(miB)
        deadB = mxB == neg
        bx1 = jnp.where(deadB, f0x1, bx1)
        by1 = jnp.where(deadB, f0y1, by1)
        bx2 = jnp.where(deadB, f0x2, bx2)
        by2 = jnp.where(deadB, f0y2, by2)
        bar = jnp.where(deadB, f0ar, bar)
        pxx1 = jnp.maximum(ax1, bx1)
        pyy1 = jnp.maximum(ay1, by1)
        pxx2 = jnp.minimum(ax2, bx2)
        pyy2 = jnp.minimum(ay2, by2)
        piw = jnp.maximum(0.0, pxx2 - pxx1 + 1.0)
        pih = jnp.maximum(0.0, pyy2 - pyy1 + 1.0)
        pinter = piw * pih
        piou = pinter / (aar + bar - pinter)
        pairok = jnp.logical_and(piou <= _NMS_THRESH,
                                 k < _POST_NMS - 1)
        out_ref[pl.ds(k, 1), :] = _rowof(ax1, ay1, ax2, ay2)
        kb = jnp.where(pairok, k + 1, jnp.int32(_POST_NMS))
        out_ref[pl.ds(kb, 1), :] = _rowof(bx1, by1, bx2, by2)
        axx1 = jnp.maximum(ax1, x1)
        ayy1 = jnp.maximum(ay1, y1)
        axx2 = jnp.minimum(ax2, x2)
        ayy2 = jnp.minimum(ay2, y2)
        aiw = jnp.maximum(0.0, axx2 - axx1 + 1.0)
        aih = jnp.maximum(0.0, ayy2 - ayy1 + 1.0)
        ainter = aiw * aih
        aiou = ainter / (aar + areas - ainter)
        bxx1 = jnp.maximum(bx1, x1)
        byy1 = jnp.maximum(by1, y1)
        bxx2 = jnp.minimum(bx2, x2)
        byy2 = jnp.minimum(by2, y2)
        biw = jnp.maximum(0.0, bxx2 - bxx1 + 1.0)
        bih = jnp.maximum(0.0, byy2 - byy1 + 1.0)
        binter = biw * bih
        biou = binter / (bar + areas - binter)
        supp = (aiou > _NMS_THRESH) | jnp.logical_and(pairok,
                                                      biou > _NMS_THRESH)
        s_ref[:] = jnp.where(supp, neg, s)
        k = k + jnp.where(pairok, jnp.int32(2), jnp.int32(1))
        return (k, f0x1, f0y1, f0x2, f0y2, f0ar)

    zero = jnp.float32(0.0)
    jax.lax.while_loop(nms_cond, nms_trip,
                       (jnp.int32(0), zero, zero, zero, zero, zero))


def _plane(v, pad):
    return jnp.concatenate([v, jnp.zeros((pad,), v.dtype)]).reshape(_ROWS, _LANES)


def kernel(rpn_cls_score, rpn_bbox_pred):
    cls = rpn_cls_score.reshape(_N, 2)
    deltas = rpn_bbox_pred.reshape(_N, 4)
    pad = _NPAD - _N
    args = (_plane(cls[:, 0], pad), _plane(cls[:, 1], pad),
            _plane(deltas[:, 0], pad), _plane(deltas[:, 1], pad),
            _plane(deltas[:, 2], pad), _plane(deltas[:, 3], pad),
            _AW, _AH, _ACX, _ACY)
    res = pl.pallas_call(
        _body,
        out_shape=jax.ShapeDtypeStruct((_OUT_ROWS, _LANES), jnp.float32),
        scratch_shapes=[pltpu.VMEM((_ROWS, _LANES), jnp.float32)
                        for _ in range(6)],
    )(*args)
    boxes = res[:_POST_NMS, :4]
    return jnp.concatenate(
        [jnp.zeros((_POST_NMS, 1), jnp.float32), boxes], axis=1)


# speedup vs baseline: 18.9557x; 1.3766x over previous
import numpy as np
import jax
import jax.numpy as jnp
from jax.experimental import pallas as pl
from jax.experimental.pallas import tpu as pltpu

_IM_H, _IM_W = 800, 800
_FEAT_STRIDE = 16
_H, _W = _IM_H // _FEAT_STRIDE, _IM_W // _FEAT_STRIDE
_A = 9
_N = _H * _W * _A
_LANES = 128
_ROWS = 176
_NPAD = _ROWS * _LANES
_PRE_NMS = 6000
_POST_NMS = 300
_NMS_THRESH = 0.7
_MIN_SIZE = 16.0
_OUT_ROWS = 304


def _np_base_anchors():
    scales = np.array([8.0, 16.0, 32.0])
    ratios = np.array([0.5, 1.0, 2.0])
    w, h, xc, yc = 16.0, 16.0, 7.5, 7.5
    size = w * h
    size_ratios = size / ratios
    ws = np.round(np.sqrt(size_ratios))
    hs = np.round(ws * ratios)
    ratio_anchors = np.stack([xc - 0.5 * (ws - 1), yc - 0.5 * (hs - 1),
                              xc + 0.5 * (ws - 1), yc + 0.5 * (hs - 1)], axis=1)
    out = []
    for a in ratio_anchors:
        aw = a[2] - a[0] + 1.0
        ah = a[3] - a[1] + 1.0
        axc = a[0] + 0.5 * (aw - 1)
        ayc = a[1] + 0.5 * (ah - 1)
        ws2 = aw * scales
        hs2 = ah * scales
        out.append(np.stack([axc - 0.5 * (ws2 - 1), ayc - 0.5 * (hs2 - 1),
                             axc + 0.5 * (ws2 - 1), ayc + 0.5 * (hs2 - 1)], axis=1))
    return np.concatenate(out, axis=0).astype(np.float32)


def _np_anchor_planes():
    base = _np_base_anchors()
    sx = np.arange(_W) * _FEAT_STRIDE
    sy = np.arange(_H) * _FEAT_STRIDE
    SX, SY = np.meshgrid(sx, sy)
    shifts = np.stack([SX.ravel(), SY.ravel(), SX.ravel(), SY.ravel()],
                      axis=1).astype(np.float32)
    anchors = (shifts[:, None, :] + base[None, :, :]).reshape(-1, 4)
    widths = anchors[:, 2] - anchors[:, 0] + np.float32(1.0)
    heights = anchors[:, 3] - anchors[:, 1] + np.float32(1.0)
    ctr_x = anchors[:, 0] + np.float32(0.5) * widths
    ctr_y = anchors[:, 1] + np.float32(0.5) * heights

    def plane(v, fill):
        out = np.full((_NPAD,), fill, np.float32)
        out[:_N] = v
        return out.reshape(_ROWS, _LANES)

    return (plane(widths, 1.0), plane(heights, 1.0),
            plane(ctr_x, 0.0), plane(ctr_y, 0.0))


_AW, _AH, _ACX, _ACY = _np_anchor_planes()


def _body(y_ref, z_ref,
          aw_ref, ah_ref, acx_ref, acy_ref, out_ref,
          s_ref, x1_s, y1_s, x2_s, y2_s, ar_s):
    Y = y_ref[:]
    r2 = jax.lax.broadcasted_iota(jnp.int32, (2 * _LANES, _LANES), 0)
    c2 = jax.lax.broadcasted_iota(jnp.int32, (2 * _LANES, _LANES), 1)
    a = jnp.dot(Y, (r2 == 2 * c2).astype(jnp.float32),
                preferred_element_type=jnp.float32)
    b = jnp.dot(Y, (r2 == 2 * c2 + 1).astype(jnp.float32),
                preferred_element_type=jnp.float32)
    Z = z_ref[:]
    r4 = jax.lax.broadcasted_iota(jnp.int32, (4 * _LANES, _LANES), 0)
    c4 = jax.lax.broadcasted_iota(jnp.int32, (4 * _LANES, _LANES), 1)
    dx = jnp.dot(Z, (r4 == 4 * c4).astype(jnp.float32),
                 preferred_element_type=jnp.float32)
    dy = jnp.dot(Z, (r4 == 4 * c4 + 1).astype(jnp.float32),
                 preferred_element_type=jnp.float32)
    dw = jnp.dot(Z, (r4 == 4 * c4 + 2).astype(jnp.float32),
                 preferred_element_type=jnp.float32)
    dh = jnp.dot(Z, (r4 == 4 * c4 + 3).astype(jnp.float32),
                 preferred_element_type=jnp.float32)

    m = jnp.maximum(a, b)
    ea = jnp.exp(a - m)
    eb = jnp.exp(b - m)
    sc = eb / (ea + eb)

    wdt = aw_ref[:]
    hgt = ah_ref[:]
    cx = acx_ref[:]
    cy = acy_ref[:]
    pcx = dx * wdt + cx
    pcy = dy * hgt + cy
    pw = jnp.exp(dw) * wdt
    ph = jnp.exp(dh) * hgt
    x1 = jnp.clip(pcx - 0.5 * pw, 0.0, _IM_W - 1.0)
    y1 = jnp.clip(pcy - 0.5 * ph, 0.0, _IM_H - 1.0)
    x2 = jnp.clip(pcx + 0.5 * pw, 0.0, _IM_W - 1.0)
    y2 = jnp.clip(pcy + 0.5 * ph, 0.0, _IM_H - 1.0)
    ws = x2 - x1 + 1.0
    hs = y2 - y1 + 1.0
    valid = (ws >= _MIN_SIZE) & (hs >= _MIN_SIZE)
    sc = jnp.where(valid, sc, -1e9)

    rows = jax.lax.broadcasted_iota(jnp.int32, (_ROWS, _LANES), 0)
    cols = jax.lax.broadcasted_iota(jnp.int32, (_ROWS, _LANES), 1)
    flat = rows * _LANES + cols
    sc = jnp.where(flat < _N, sc, -jnp.inf)

    kraw = jax.lax.bitcast_convert_type(sc, jnp.int32)
    keys = jnp.where(kraw < 0, kraw ^ jnp.int32(0x7FFFFFFF), kraw)

    cnt_pos = jnp.sum((keys >= 0).astype(jnp.int32))
    t0 = jnp.where(cnt_pos >= _PRE_NMS, jnp.int32(0), jnp.int32(-2147483648))

    def bs_body(k, t):
        cand = t + jnp.left_shift(jnp.int32(1), 30 - k)
        cnt = jnp.sum((keys >= cand).astype(jnp.int32))
        return jnp.where(cnt >= _PRE_NMS, cand, t)

    thr = jax.lax.fori_loop(0, 31, bs_body, t0)

    s_ref[:] = jnp.where(keys >= thr, sc, -jnp.inf)
    areas = ws * hs
    x1_s[:] = x1
    y1_s[:] = y1
    x2_s[:] = x2
    y2_s[:] = y2
    ar_s[:] = areas
    neg = jnp.float32(-jnp.inf)
    li = jax.lax.broadcasted_iota(jnp.int32, (1, _LANES), 1)

    imax = jnp.int32(2147483647)

    def _extract(mi):
        ri = jax.lax.shift_right_logical(mi, 7)
        ci = jax.lax.bitwise_and(mi, 127)
        oh = li == ci
        px1 = jnp.sum(jnp.where(oh, x1_s[pl.ds(ri, 1), :], 0.0))
        py1 = jnp.sum(jnp.where(oh, y1_s[pl.ds(ri, 1), :], 0.0))
        px2 = jnp.sum(jnp.where(oh, x2_s[pl.ds(ri, 1), :], 0.0))
        py2 = jnp.sum(jnp.where(oh, y2_s[pl.ds(ri, 1), :], 0.0))
        par = jnp.sum(jnp.where(oh, ar_s[pl.ds(ri, 1), :], 0.0))
        return px1, py1, px2, py2, par

    def _rowof(px1, py1, px2, py2):
        return jnp.where(li == 0, px1,
                         jnp.where(li == 1, py1,
                                   jnp.where(li == 2, px2,
                                             jnp.where(li == 3, py2, 0.0))))

    def nms_cond(carry):
        return carry[0] < _POST_NMS

    def nms_trip(carry):
        k, f0x1, f0y1, f0x2, f0y2, f0ar = carry
        s = s_ref[:]
        mx = jnp.max(s)
        mi = jnp.min(jnp.where(s == mx, flat, imax))
        ax1, ay1, ax2, ay2, aar = _extract(mi)
        first = k == 0
        f0x1 = jnp.where(first, ax1, f0x1)
        f0y1 = jnp.where(first, ay1, f0y1)
        f0x2 = jnp.where(first, ax2, f0x2)
        f0y2 = jnp.where(first, ay2, f0y2)
        f0ar = jnp.where(first, aar, f0ar)
        deadA = mx == neg
        ax1 = jnp.where(deadA, f0x1, ax1)
        ay1 = jnp.where(deadA, f0y1, ay1)
        ax2 = jnp.where(deadA, f0x2, ax2)
        ay2 = jnp.where(deadA, f0y2, ay2)
        aar = jnp.where(deadA, f0ar, aar)
        sB = jnp.where(flat == mi, neg, s)
        mxB = jnp.max(sB)
        miB = jnp.min(jnp.where(sB == mxB, flat, imax))
        bx1, by1, bx2, by2, bar = _extract(miB)
        deadB = mxB == neg
        bx1 = jnp.where(deadB, f0x1, bx1)
        by1 = jnp.where(deadB, f0y1, by1)
        bx2 = jnp.where(deadB, f0x2, bx2)
        by2 = jnp.where(deadB, f0y2, by2)
        bar = jnp.where(deadB, f0ar, bar)
        pxx1 = jnp.maximum(ax1, bx1)
        pyy1 = jnp.maximum(ay1, by1)
        pxx2 = jnp.minimum(ax2, bx2)
        pyy2 = jnp.minimum(ay2, by2)
        piw = jnp.maximum(0.0, pxx2 - pxx1 + 1.0)
        pih = jnp.maximum(0.0, pyy2 - pyy1 + 1.0)
        pinter = piw * pih
        piou = pinter / (aar + bar - pinter)
        pairok = jnp.logical_and(piou <= _NMS_THRESH,
                                 k < _POST_NMS - 1)
        out_ref[pl.ds(k, 1), :] = _rowof(ax1, ay1, ax2, ay2)
        kb = jnp.where(pairok, k + 1, jnp.int32(_POST_NMS))
        out_ref[pl.ds(kb, 1), :] = _rowof(bx1, by1, bx2, by2)
        axx1 = jnp.maximum(ax1, x1)
        ayy1 = jnp.maximum(ay1, y1)
        axx2 = jnp.minimum(ax2, x2)
        ayy2 = jnp.minimum(ay2, y2)
        aiw = jnp.maximum(0.0, axx2 - axx1 + 1.0)
        aih = jnp.maximum(0.0, ayy2 - ayy1 + 1.0)
        ainter = aiw * aih
        aiou = ainter / (aar + areas - ainter)
        bxx1 = jnp.maximum(bx1, x1)
        byy1 = jnp.maximum(by1, y1)
        bxx2 = jnp.minimum(bx2, x2)
        byy2 = jnp.minimum(by2, y2)
        biw = jnp.maximum(0.0, bxx2 - bxx1 + 1.0)
        bih = jnp.maximum(0.0, byy2 - byy1 + 1.0)
        binter = biw * bih
        biou = binter / (bar + areas - binter)
        supp = (aiou > _NMS_THRESH) | jnp.logical_and(pairok,
                                                      biou > _NMS_THRESH)
        s_ref[:] = jnp.where(supp, neg, s)
        k = k + jnp.where(pairok, jnp.int32(2), jnp.int32(1))
        return (k, f0x1, f0y1, f0x2, f0y2, f0ar)

    zero = jnp.float32(0.0)
    jax.lax.while_loop(nms_cond, nms_trip,
                       (jnp.int32(0), zero, zero, zero, zero, zero))


def _prep(rpn_cls_score, rpn_bbox_pred):
    y = jnp.concatenate([rpn_cls_score.reshape(-1),
                         jnp.zeros((2 * _NPAD - 2 * _N,), jnp.float32)])
    z = jnp.concatenate([rpn_bbox_pred.reshape(-1),
                         jnp.zeros((4 * _NPAD - 4 * _N,), jnp.float32)])
    return (y.reshape(_ROWS, 2 * _LANES), z.reshape(_ROWS, 4 * _LANES),
            _AW, _AH, _ACX, _ACY)


def kernel(rpn_cls_score, rpn_bbox_pred):
    res = pl.pallas_call(
        _body,
        out_shape=jax.ShapeDtypeStruct((_OUT_ROWS, _LANES), jnp.float32),
        scratch_shapes=[pltpu.VMEM((_ROWS, _LANES), jnp.float32)
                        for _ in range(6)],
    )(*_prep(rpn_cls_score, rpn_bbox_pred))
    boxes = res[:_POST_NMS, :4]
    return jnp.concatenate(
        [jnp.zeros((_POST_NMS, 1), jnp.float32), boxes], axis=1)
